# baseline (speedup 1.0000x reference)
import functools

import jax
import jax.numpy as jnp
from jax import lax
from jax.experimental import pallas as pl
from jax.experimental.pallas import tpu as pltpu
from jax.experimental.pallas import tpu_sc as plsc

N_NODES = 10000
N_EDGES = 160000
CHUNK = 100
KEEP = 50


def _mm_body(h_ref, w_ref, b_ref, o_ref, *, act):
    h = h_ref[...]
    if act:
        h = jnp.maximum(h + b_ref[...], 0.0)
    o_ref[...] = jnp.dot(h, w_ref[...], preferred_element_type=jnp.float32)


def _matmul(h, w, b, act, rows_blk=400):
    n, din = h.shape
    dout = w.shape[1]
    grid = (n // rows_blk,)
    return pl.pallas_call(
        functools.partial(_mm_body, act=act),
        grid=grid,
        in_specs=[
            pl.BlockSpec((rows_blk, din), lambda i: (i, 0)),
            pl.BlockSpec((din, dout), lambda i: (0, 0)),
            pl.BlockSpec((1, din), lambda i: (0, 0)),
        ],
        out_specs=pl.BlockSpec((rows_blk, dout), lambda i: (i, 0)),
        out_shape=jax.ShapeDtypeStruct((n, dout), jnp.float32),
    )(h, w, b.reshape(1, -1))


def _epi_body(x_ref, b_ref, o_ref):
    o_ref[...] = jnp.tanh(x_ref[...] + b_ref[...]) + 1.0


def _epilogue(x, b, rows_blk=200):
    n, d = x.shape
    return pl.pallas_call(
        _epi_body,
        grid=(n // rows_blk,),
        in_specs=[
            pl.BlockSpec((rows_blk, d), lambda i: (i, 0)),
            pl.BlockSpec((1, d), lambda i: (0, 0)),
        ],
        out_specs=pl.BlockSpec((rows_blk, d), lambda i: (i, 0)),
        out_shape=jax.ShapeDtypeStruct((n, d), jnp.float32),
    )(x, b.reshape(1, -1))


_NS = 16
_NW = 32
_TILE_ROWS = 320
_E_PAD = N_EDGES + 512


_GC = 64
_MB = 8


def _make_spmm_sc(D):
    G = D // 16
    mesh = plsc.VectorSubcoreMesh(core_axis_name="c", subcore_axis_name="s")

    def body(sup_hbm, src_hbm, dst_hbm, w_hbm, bnd_hbm, out_hbm,
             acc_v, src_v, dst_v, w_v, buf0, buf1, bnd_v, sem0, sem1):
        c = lax.axis_index("c")
        s = lax.axis_index("s")
        wid = c * _NS + s
        lo = wid * _TILE_ROWS

        pltpu.sync_copy(bnd_hbm, bnd_v)

        zv = jnp.zeros((16,), jnp.float32)

        def zbody(r, zc):
            for g in range(G):
                acc_v[r, pl.ds(g * 16, 16)] = zv
            return zc

        lax.fori_loop(0, _TILE_ROWS, zbody, 0)

        widx = jnp.zeros((16,), jnp.int32) + wid
        start_e = plsc.load_gather(bnd_v, [widx])[0]
        end_e = plsc.load_gather(bnd_v, [widx + 1])[0]
        a = (start_e // _GC) * _GC
        nmeta = (end_e - a + _GC * _MB - 1) // (_GC * _MB)

        def issue(kk, buf, sem):
            return pltpu.async_copy(
                sup_hbm.at[src_v.at[pl.ds(kk * _GC, _GC)]], buf, sem)

        def wait(kk, buf, sem):
            pltpu.make_async_copy(
                sup_hbm.at[src_v.at[pl.ds(kk * _GC, _GC)]], buf, sem).wait()

        def process(kk, buf):
            @plsc.parallel_loop(0, G, step=1, unroll=4)
            def _(g):
                fsl = pl.ds(g * 16, 16)
                for j in range(4):
                    dl = dst_v[pl.ds(kk * _GC + j * 16, 16)]
                    ww = w_v[pl.ds(kk * _GC + j * 16, 16)]
                    for jj in range(16):
                        plsc.addupdate(acc_v.at[dl[jj], fsl],
                                       buf[j * 16 + jj, fsl] * ww[jj])

        def mblock(m, carry):
            e0 = a + m * (_GC * _MB)
            pltpu.sync_copy(src_hbm.at[pl.ds(e0, _GC * _MB)], src_v)
            pltpu.sync_copy(dst_hbm.at[pl.ds(e0, _GC * _MB)],
                            dst_v.at[pl.ds(0, _GC * _MB)])
            pltpu.sync_copy(w_hbm.at[pl.ds(e0, _GC * _MB)],
                            w_v.at[pl.ds(0, _GC * _MB)])
            for i in range(_GC * _MB // 16):
                sl = pl.ds(i * 16, 16)
                dd = dst_v[sl]
                ok = (dd >= lo) & (dd < lo + _TILE_ROWS)
                w_v[sl] = jnp.where(ok, w_v[sl], 0.0)
                dst_v[sl] = jnp.clip(dd - lo, 0, _TILE_ROWS - 1)
            issue(0, buf0, sem0)
            issue(1, buf1, sem1)

            def pair(p, pc):
                wait(2 * p, buf0, sem0)
                process(2 * p, buf0)

                @pl.when(2 * p + 2 < _MB)
                def _():
                    issue(2 * p + 2, buf0, sem0)

                wait(2 * p + 1, buf1, sem1)
                process(2 * p + 1, buf1)

                @pl.when(2 * p + 3 < _MB)
                def _():
                    issue(2 * p + 3, buf1, sem1)

                return pc

            lax.fori_loop(0, _MB // 2, pair, 0)
            return carry

        lax.fori_loop(0, nmeta, mblock, 0)

        last = N_NODES - (_NW - 1) * _TILE_ROWS

        @pl.when(wid < _NW - 1)
        def _():
            pltpu.sync_copy(acc_v, out_hbm.at[pl.ds(lo, _TILE_ROWS)])

        @pl.when(wid == _NW - 1)
        def _():
            pltpu.sync_copy(acc_v.at[pl.ds(0, last)],
                            out_hbm.at[pl.ds(lo, last)])

    return pl.kernel(
        body,
        out_type=jax.ShapeDtypeStruct((N_NODES, D), jnp.float32),
        mesh=mesh,
        compiler_params=pltpu.CompilerParams(needs_layout_passes=False),
        scratch_types=[
            pltpu.VMEM((_TILE_ROWS, D), jnp.float32),
            pltpu.VMEM((_GC * _MB,), jnp.int32),
            pltpu.VMEM((_GC * _MB + 16,), jnp.int32),
            pltpu.VMEM((_GC * _MB + 16,), jnp.float32),
            pltpu.VMEM((_GC, D), jnp.float32),
            pltpu.VMEM((_GC, D), jnp.float32),
            pltpu.VMEM((40,), jnp.int32),
            pltpu.SemaphoreType.DMA,
            pltpu.SemaphoreType.DMA,
        ],
    )


_SPMM_SC = {256: _make_spmm_sc(256), 128: _make_spmm_sc(128)}


def _spmm(support, src, dst, w, bound):
    return _SPMM_SC[support.shape[1]](support, src, dst, w, bound)


def kernel(x, edge_index, edge_weight, num_remain,
           W1, b1, W2, b2, W3, b3, W4, b4, W5, b5, W6, b6, W7, b7, W8, b8):
    src = edge_index[0]
    dst = edge_index[1]
    order = jnp.argsort(dst)
    pad = _E_PAD - N_EDGES
    src_s = jnp.concatenate([src[order].astype(jnp.int32),
                             jnp.zeros((pad,), jnp.int32)])
    dst_s = jnp.concatenate([dst[order].astype(jnp.int32),
                             jnp.zeros((pad,), jnp.int32)])
    w_s = jnp.concatenate([edge_weight[order], jnp.zeros((pad,), jnp.float32)])
    bnds = jnp.searchsorted(
        dst_s[:N_EDGES], jnp.arange(_NW + 1, dtype=jnp.int32) * _TILE_ROWS
    ).astype(jnp.int32)
    bound = jnp.concatenate([bnds, jnp.zeros((40 - _NW - 1,), jnp.int32)])

    Ws = [W1, W2, W3, W4, W5, W6, W7, W8]
    bs = [b1, b2, b3, b4, b5, b6, b7, b8]

    h = x
    for l in range(8):
        support = _matmul(h, Ws[l], bs[l - 1] if l > 0 else bs[0], act=(l > 0))
        h = _spmm(support, src_s, dst_s, w_s, bound)

    n_chunks = N_NODES // CHUNK
    reshaped = h.reshape(n_chunks, CHUNK, h.shape[-1])
    start = num_remain - KEEP
    sliced = lax.dynamic_slice_in_dim(reshaped, start, KEEP, axis=1)
    flat = sliced.reshape(n_chunks * KEEP, h.shape[-1])
    out = _epilogue(flat, bs[7])
    return out.reshape(n_chunks, KEEP, h.shape[-1])


# speedup vs baseline: 2.5401x; 2.5401x over previous
import functools

import jax
import jax.numpy as jnp
from jax import lax
from jax.experimental import pallas as pl
from jax.experimental.pallas import tpu as pltpu
from jax.experimental.pallas import tpu_sc as plsc

N_NODES = 10000
N_EDGES = 160000
CHUNK = 100
KEEP = 50


def _mm_body(h_ref, w_ref, b_ref, o_ref, *, act):
    h = h_ref[...]
    if act:
        h = jnp.maximum(h + b_ref[...], 0.0)
    o_ref[...] = jnp.dot(h, w_ref[...], preferred_element_type=jnp.float32)


def _matmul(h, w, b, act, rows_blk=400):
    n, din = h.shape
    dout = w.shape[1]
    grid = (n // rows_blk,)
    return pl.pallas_call(
        functools.partial(_mm_body, act=act),
        grid=grid,
        in_specs=[
            pl.BlockSpec((rows_blk, din), lambda i: (i, 0)),
            pl.BlockSpec((din, dout), lambda i: (0, 0)),
            pl.BlockSpec((1, din), lambda i: (0, 0)),
        ],
        out_specs=pl.BlockSpec((rows_blk, dout), lambda i: (i, 0)),
        out_shape=jax.ShapeDtypeStruct((n, dout), jnp.float32),
    )(h, w, b.reshape(1, -1))


def _epi_body(x_ref, b_ref, o_ref):
    o_ref[...] = jnp.tanh(x_ref[...] + b_ref[...]) + 1.0


def _epilogue(x, b, rows_blk=200):
    n, d = x.shape
    return pl.pallas_call(
        _epi_body,
        grid=(n // rows_blk,),
        in_specs=[
            pl.BlockSpec((rows_blk, d), lambda i: (i, 0)),
            pl.BlockSpec((1, d), lambda i: (0, 0)),
        ],
        out_specs=pl.BlockSpec((rows_blk, d), lambda i: (i, 0)),
        out_shape=jax.ShapeDtypeStruct((n, d), jnp.float32),
    )(x, b.reshape(1, -1))


_NS = 16
_NW = 32
_TILE_ROWS = 320
_E_PAD = N_EDGES + 512


_GC = 64
_MB = 8


def _make_spmm_sc(D):
    G = D // 16
    mesh = plsc.VectorSubcoreMesh(core_axis_name="c", subcore_axis_name="s")

    def body(sup_hbm, src_hbm, dst_hbm, w_hbm, bnd_hbm, out_hbm,
             acc_v, src_v, dst_v, w_v, buf0, buf1, bnd_v, sem0, sem1):
        c = lax.axis_index("c")
        s = lax.axis_index("s")
        wid = c * _NS + s
        lo = wid * _TILE_ROWS

        pltpu.sync_copy(bnd_hbm, bnd_v)

        zv = jnp.zeros((16,), jnp.float32)

        def zbody(r, zc):
            for g in range(G):
                acc_v[r, pl.ds(g * 16, 16)] = zv
            return zc

        lax.fori_loop(0, _TILE_ROWS, zbody, 0)

        widx = jnp.zeros((16,), jnp.int32) + wid
        start_e = plsc.load_gather(bnd_v, [widx])[0]
        end_e = plsc.load_gather(bnd_v, [widx + 1])[0]
        a = (start_e // _GC) * _GC
        nmeta = (end_e - a + _GC * _MB - 1) // (_GC * _MB)

        def issue(kk, buf, sem):
            return pltpu.async_copy(
                sup_hbm.at[src_v.at[pl.ds(kk * _GC, _GC)]], buf, sem)

        def wait(kk, buf, sem):
            pltpu.make_async_copy(
                sup_hbm.at[src_v.at[pl.ds(kk * _GC, _GC)]], buf, sem).wait()

        def process(kk, buf):
            @plsc.parallel_loop(0, G, step=1, unroll=2)
            def _(g):
                fsl = pl.ds(g * 16, 16)
                for j in range(4):
                    dl = dst_v[pl.ds(kk * _GC + j * 16, 16)]
                    ww = w_v[pl.ds(kk * _GC + j * 16, 16)]
                    for jj in range(16):
                        plsc.addupdate(acc_v.at[dl[jj], fsl],
                                       buf[j * 16 + jj, fsl] * ww[jj])

        def mblock(m, carry):
            e0 = a + m * (_GC * _MB)
            pltpu.sync_copy(src_hbm.at[pl.ds(e0, _GC * _MB)], src_v)
            pltpu.sync_copy(dst_hbm.at[pl.ds(e0, _GC * _MB)],
                            dst_v.at[pl.ds(0, _GC * _MB)])
            pltpu.sync_copy(w_hbm.at[pl.ds(e0, _GC * _MB)],
                            w_v.at[pl.ds(0, _GC * _MB)])
            for i in range(_GC * _MB // 16):
                sl = pl.ds(i * 16, 16)
                dd = dst_v[sl]
                ok = (dd >= lo) & (dd < lo + _TILE_ROWS)
                w_v[sl] = jnp.where(ok, w_v[sl], 0.0)
                dst_v[sl] = jnp.clip(dd - lo, 0, _TILE_ROWS - 1)
            issue(0, buf0, sem0)
            issue(1, buf1, sem1)

            def pair(p, pc):
                wait(2 * p, buf0, sem0)
                process(2 * p, buf0)

                @pl.when(2 * p + 2 < _MB)
                def _():
                    issue(2 * p + 2, buf0, sem0)

                wait(2 * p + 1, buf1, sem1)
                process(2 * p + 1, buf1)

                @pl.when(2 * p + 3 < _MB)
                def _():
                    issue(2 * p + 3, buf1, sem1)

                return pc

            lax.fori_loop(0, _MB // 2, pair, 0)
            return carry

        lax.fori_loop(0, nmeta, mblock, 0)

        last = N_NODES - (_NW - 1) * _TILE_ROWS

        @pl.when(wid < _NW - 1)
        def _():
            pltpu.sync_copy(acc_v, out_hbm.at[pl.ds(lo, _TILE_ROWS)])

        @pl.when(wid == _NW - 1)
        def _():
            pltpu.sync_copy(acc_v.at[pl.ds(0, last)],
                            out_hbm.at[pl.ds(lo, last)])

    return pl.kernel(
        body,
        out_type=jax.ShapeDtypeStruct((N_NODES, D), jnp.float32),
        mesh=mesh,
        compiler_params=pltpu.CompilerParams(needs_layout_passes=False),
        scratch_types=[
            pltpu.VMEM((_TILE_ROWS, D), jnp.float32),
            pltpu.VMEM((_GC * _MB,), jnp.int32),
            pltpu.VMEM((_GC * _MB + 16,), jnp.int32),
            pltpu.VMEM((_GC * _MB + 16,), jnp.float32),
            pltpu.VMEM((_GC, D), jnp.float32),
            pltpu.VMEM((_GC, D), jnp.float32),
            pltpu.VMEM((40,), jnp.int32),
            pltpu.SemaphoreType.DMA,
            pltpu.SemaphoreType.DMA,
        ],
    )


_SPMM_SC = {256: _make_spmm_sc(256), 128: _make_spmm_sc(128)}


def _spmm(support, src, dst, w, bound):
    return _SPMM_SC[support.shape[1]](support, src, dst, w, bound)


def kernel(x, edge_index, edge_weight, num_remain,
           W1, b1, W2, b2, W3, b3, W4, b4, W5, b5, W6, b6, W7, b7, W8, b8):
    src = edge_index[0]
    dst = edge_index[1]
    order = jnp.argsort(dst)
    pad = _E_PAD - N_EDGES
    src_s = jnp.concatenate([src[order].astype(jnp.int32),
                             jnp.zeros((pad,), jnp.int32)])
    dst_s = jnp.concatenate([dst[order].astype(jnp.int32),
                             jnp.zeros((pad,), jnp.int32)])
    w_s = jnp.concatenate([edge_weight[order], jnp.zeros((pad,), jnp.float32)])
    bnds = jnp.searchsorted(
        dst_s[:N_EDGES], jnp.arange(_NW + 1, dtype=jnp.int32) * _TILE_ROWS
    ).astype(jnp.int32)
    bound = jnp.concatenate([bnds, jnp.zeros((40 - _NW - 1,), jnp.int32)])

    Ws = [W1, W2, W3, W4, W5, W6, W7, W8]
    bs = [b1, b2, b3, b4, b5, b6, b7, b8]

    h = x
    for l in range(8):
        support = _matmul(h, Ws[l], bs[l - 1] if l > 0 else bs[0], act=(l > 0))
        h = _spmm(support, src_s, dst_s, w_s, bound)

    n_chunks = N_NODES // CHUNK
    reshaped = h.reshape(n_chunks, CHUNK, h.shape[-1])
    start = num_remain - KEEP
    sliced = lax.dynamic_slice_in_dim(reshaped, start, KEEP, axis=1)
    flat = sliced.reshape(n_chunks * KEEP, h.shape[-1])
    out = _epilogue(flat, bs[7])
    return out.reshape(n_chunks, KEEP, h.shape[-1])


# speedup vs baseline: 5.2126x; 2.0521x over previous
import functools

import jax
import jax.numpy as jnp
from jax import lax
from jax.experimental import pallas as pl
from jax.experimental.pallas import tpu as pltpu
from jax.experimental.pallas import tpu_sc as plsc

N_NODES = 10000
N_EDGES = 160000
CHUNK = 100
KEEP = 50


def _mm_body(h_ref, w_ref, b_ref, o_ref, *, act):
    h = h_ref[...]
    if act:
        h = jnp.maximum(h + b_ref[...], 0.0)
    o_ref[...] = jnp.dot(h, w_ref[...], preferred_element_type=jnp.float32)


def _matmul(h, w, b, act, rows_blk=400):
    n, din = h.shape
    dout = w.shape[1]
    grid = (n // rows_blk,)
    return pl.pallas_call(
        functools.partial(_mm_body, act=act),
        grid=grid,
        in_specs=[
            pl.BlockSpec((rows_blk, din), lambda i: (i, 0)),
            pl.BlockSpec((din, dout), lambda i: (0, 0)),
            pl.BlockSpec((1, din), lambda i: (0, 0)),
        ],
        out_specs=pl.BlockSpec((rows_blk, dout), lambda i: (i, 0)),
        out_shape=jax.ShapeDtypeStruct((n, dout), jnp.float32),
    )(h, w, b.reshape(1, -1))


def _epi_body(x_ref, b_ref, o_ref):
    o_ref[...] = jnp.tanh(x_ref[...] + b_ref[...]) + 1.0


def _epilogue(x, b, rows_blk=200):
    n, d = x.shape
    return pl.pallas_call(
        _epi_body,
        grid=(n // rows_blk,),
        in_specs=[
            pl.BlockSpec((rows_blk, d), lambda i: (i, 0)),
            pl.BlockSpec((1, d), lambda i: (0, 0)),
        ],
        out_specs=pl.BlockSpec((rows_blk, d), lambda i: (i, 0)),
        out_shape=jax.ShapeDtypeStruct((n, d), jnp.float32),
    )(x, b.reshape(1, -1))


_NS = 16
_NW = 32
_TILE_ROWS = 320
_E_PAD = N_EDGES + 512


_GC = 64
_MB = 8


def _make_spmm_sc(D):
    G = D // 16
    mesh = plsc.VectorSubcoreMesh(core_axis_name="c", subcore_axis_name="s")

    def body(sup_hbm, src_hbm, dst_hbm, w_hbm, bnd_hbm, out_hbm,
             acc_v, src_v, dst_v, w_v, buf0, buf1, bnd_v, sem0, sem1):
        c = lax.axis_index("c")
        s = lax.axis_index("s")
        wid = c * _NS + s
        lo = wid * _TILE_ROWS

        pltpu.sync_copy(bnd_hbm, bnd_v)

        zv = jnp.zeros((16,), jnp.float32)

        def zbody(r, zc):
            for g in range(G):
                acc_v[r, pl.ds(g * 16, 16)] = zv
            return zc

        lax.fori_loop(0, _TILE_ROWS, zbody, 0)

        widx = jnp.zeros((16,), jnp.int32) + wid
        start_e = plsc.load_gather(bnd_v, [widx])[0]
        end_e = plsc.load_gather(bnd_v, [widx + 1])[0]
        a = (start_e // _GC) * _GC
        nmeta = (end_e - a + _GC * _MB - 1) // (_GC * _MB)

        def issue(kk, buf, sem):
            return pltpu.async_copy(
                sup_hbm.at[src_v.at[pl.ds(kk * _GC, _GC)]], buf, sem)

        def wait(kk, buf, sem):
            pltpu.make_async_copy(
                sup_hbm.at[src_v.at[pl.ds(kk * _GC, _GC)]], buf, sem).wait()

        def process(kk, buf):
            base = kk * _GC
            prev0 = dst_v[pl.ds(base, 16)][0]
            init = (prev0,) + tuple(
                jnp.zeros((16,), jnp.float32) for _ in range(G))

            @plsc.parallel_loop(base, base + _GC, step=1, unroll=2,
                                carry=init)
            def fin(e, carry):
                prev = carry[0]
                accs = carry[1:]
                d = dst_v[pl.ds(e, 16)][0]
                w = w_v[pl.ds(e, 16)][0]
                fl = d != prev

                @pl.when(fl)
                def _():
                    for g in range(G):
                        plsc.addupdate(acc_v.at[prev, pl.ds(g * 16, 16)],
                                       accs[g])

                el = e - base
                keep = jnp.where(fl, 0.0, 1.0)
                new = []
                for g in range(G):
                    rw = buf[el, pl.ds(g * 16, 16)] * w
                    new.append(accs[g] * keep + rw)
                return (d,) + tuple(new)

            for g in range(G):
                plsc.addupdate(acc_v.at[fin[0], pl.ds(g * 16, 16)],
                               fin[1 + g])

        def mblock(m, carry):
            e0 = a + m * (_GC * _MB)
            pltpu.sync_copy(src_hbm.at[pl.ds(e0, _GC * _MB)], src_v)
            pltpu.sync_copy(dst_hbm.at[pl.ds(e0, _GC * _MB)],
                            dst_v.at[pl.ds(0, _GC * _MB)])
            pltpu.sync_copy(w_hbm.at[pl.ds(e0, _GC * _MB)],
                            w_v.at[pl.ds(0, _GC * _MB)])
            for i in range(_GC * _MB // 16):
                sl = pl.ds(i * 16, 16)
                dd = dst_v[sl]
                ok = (dd >= lo) & (dd < lo + _TILE_ROWS)
                w_v[sl] = jnp.where(ok, w_v[sl], 0.0)
                dst_v[sl] = jnp.clip(dd - lo, 0, _TILE_ROWS - 1)
            issue(0, buf0, sem0)
            issue(1, buf1, sem1)

            def pair(p, pc):
                wait(2 * p, buf0, sem0)
                process(2 * p, buf0)

                @pl.when(2 * p + 2 < _MB)
                def _():
                    issue(2 * p + 2, buf0, sem0)

                wait(2 * p + 1, buf1, sem1)
                process(2 * p + 1, buf1)

                @pl.when(2 * p + 3 < _MB)
                def _():
                    issue(2 * p + 3, buf1, sem1)

                return pc

            lax.fori_loop(0, _MB // 2, pair, 0)
            return carry

        lax.fori_loop(0, nmeta, mblock, 0)

        last = N_NODES - (_NW - 1) * _TILE_ROWS

        @pl.when(wid < _NW - 1)
        def _():
            pltpu.sync_copy(acc_v, out_hbm.at[pl.ds(lo, _TILE_ROWS)])

        @pl.when(wid == _NW - 1)
        def _():
            pltpu.sync_copy(acc_v.at[pl.ds(0, last)],
                            out_hbm.at[pl.ds(lo, last)])

    return pl.kernel(
        body,
        out_type=jax.ShapeDtypeStruct((N_NODES, D), jnp.float32),
        mesh=mesh,
        compiler_params=pltpu.CompilerParams(needs_layout_passes=False),
        scratch_types=[
            pltpu.VMEM((_TILE_ROWS, D), jnp.float32),
            pltpu.VMEM((_GC * _MB,), jnp.int32),
            pltpu.VMEM((_GC * _MB + 16,), jnp.int32),
            pltpu.VMEM((_GC * _MB + 16,), jnp.float32),
            pltpu.VMEM((_GC, D), jnp.float32),
            pltpu.VMEM((_GC, D), jnp.float32),
            pltpu.VMEM((40,), jnp.int32),
            pltpu.SemaphoreType.DMA,
            pltpu.SemaphoreType.DMA,
        ],
    )


_SPMM_SC = {256: _make_spmm_sc(256), 128: _make_spmm_sc(128)}


def _spmm(support, src, dst, w, bound):
    return _SPMM_SC[support.shape[1]](support, src, dst, w, bound)


def kernel(x, edge_index, edge_weight, num_remain,
           W1, b1, W2, b2, W3, b3, W4, b4, W5, b5, W6, b6, W7, b7, W8, b8):
    src = edge_index[0]
    dst = edge_index[1]
    order = jnp.argsort(dst)
    pad = _E_PAD - N_EDGES
    src_s = jnp.concatenate([src[order].astype(jnp.int32),
                             jnp.zeros((pad,), jnp.int32)])
    dst_s = jnp.concatenate([dst[order].astype(jnp.int32),
                             jnp.zeros((pad,), jnp.int32)])
    w_s = jnp.concatenate([edge_weight[order], jnp.zeros((pad,), jnp.float32)])
    bnds = jnp.searchsorted(
        dst_s[:N_EDGES], jnp.arange(_NW + 1, dtype=jnp.int32) * _TILE_ROWS
    ).astype(jnp.int32)
    bound = jnp.concatenate([bnds, jnp.zeros((40 - _NW - 1,), jnp.int32)])

    Ws = [W1, W2, W3, W4, W5, W6, W7, W8]
    bs = [b1, b2, b3, b4, b5, b6, b7, b8]

    h = x
    for l in range(8):
        support = _matmul(h, Ws[l], bs[l - 1] if l > 0 else bs[0], act=(l > 0))
        h = _spmm(support, src_s, dst_s, w_s, bound)

    n_chunks = N_NODES // CHUNK
    reshaped = h.reshape(n_chunks, CHUNK, h.shape[-1])
    start = num_remain - KEEP
    sliced = lax.dynamic_slice_in_dim(reshaped, start, KEEP, axis=1)
    flat = sliced.reshape(n_chunks * KEEP, h.shape[-1])
    out = _epilogue(flat, bs[7])
    return out.reshape(n_chunks, KEEP, h.shape[-1])
